# Initial kernel scaffold; baseline (speedup 1.0000x reference)
import jax
import jax.numpy as jnp
from jax.experimental import pallas as pl


def kernel(x, enc_w1, enc_b1, enc_w2, enc_b2, codebook, dec_w1, dec_b1, dec_w2, dec_b2):
    raise NotImplementedError("write your pallas kernel here")


# speedup vs baseline: 1.1612x; 1.1612x over previous
import functools

import jax
import jax.numpy as jnp
from jax.experimental import pallas as pl
from jax.experimental.pallas import tpu as pltpu


def _conv2d(x, w, b, stride, pad):
    out = jax.lax.conv_general_dilated(
        x, w, (stride, stride), ((pad, pad), (pad, pad)),
        dimension_numbers=('NCHW', 'OIHW', 'NCHW'))
    return out + b[None, :, None, None]


def _conv_transpose2d(x, w, b, stride, pad):
    k = w.shape[2]
    w_conv = jnp.transpose(jnp.flip(w, (2, 3)), (1, 0, 2, 3))
    p = k - 1 - pad
    out = jax.lax.conv_general_dilated(
        x, w_conv, (1, 1), ((p, p), (p, p)), lhs_dilation=(stride, stride),
        dimension_numbers=('NCHW', 'OIHW', 'NCHW'))
    return out + b[None, :, None, None]


def _vq_body(z_ref, cb_ref, zq_ref):
    z = z_ref[...]
    cb = cb_ref[...]
    z_norm = jnp.sum(z * z, axis=1, keepdims=True)
    cb_norm = jnp.sum(cb * cb, axis=1)[None, :]
    d = (z_norm + cb_norm) - 2.0 * jax.lax.dot_general(
        z, cb, (((1,), (1,)), ((), ())), preferred_element_type=jnp.float32)
    d_min = jnp.min(d, axis=1, keepdims=True)
    k = cb.shape[0]
    iota = jax.lax.broadcasted_iota(jnp.int32, d.shape, 1)
    masked_iota = jnp.where(d == d_min, iota, k)
    idx = jnp.min(masked_iota, axis=1, keepdims=True)
    onehot = (iota == idx).astype(jnp.float32)
    zq_ref[...] = jax.lax.dot_general(
        onehot, cb, (((1,), (0,)), ((), ())),
        preferred_element_type=jnp.float32)


@functools.partial(jax.jit, static_argnames=('blk',))
def _vq_lookup(z_flat, codebook, blk=1024):
    n, d = z_flat.shape
    k = codebook.shape[0]
    grid = n // blk
    return pl.pallas_call(
        _vq_body,
        grid=(grid,),
        in_specs=[
            pl.BlockSpec((blk, d), lambda i: (i, 0)),
            pl.BlockSpec((k, d), lambda i: (0, 0)),
        ],
        out_specs=pl.BlockSpec((blk, d), lambda i: (i, 0)),
        out_shape=jax.ShapeDtypeStruct((n, d), jnp.float32),
    )(z_flat, codebook)


def kernel(x, enc_w1, enc_b1, enc_w2, enc_b2, codebook,
           dec_w1, dec_b1, dec_w2, dec_b2):
    h = jax.nn.relu(_conv2d(x, enc_w1, enc_b1, 2, 1))
    z_e = jax.nn.relu(_conv2d(h, enc_w2, enc_b2, 2, 1))
    z_e_flat = jnp.reshape(z_e, (-1, z_e.shape[1]))
    z_q = _vq_lookup(z_e_flat, codebook).reshape(z_e.shape)
    h2 = jax.nn.relu(_conv_transpose2d(z_q, dec_w1, dec_b1, 2, 1))
    x_recon = jax.nn.sigmoid(_conv_transpose2d(h2, dec_w2, dec_b2, 2, 1))
    return (x_recon, z_q)
